# baseline (device time: 1399951 ns/iter reference)
import jax
import jax.numpy as jnp
from jax import lax
from jax.experimental import pallas as pl
from jax.experimental.pallas import tpu as pltpu

N_DEV = 16
CH = 256


def _pos_to_ring(pos):
    z = pos // 4
    k = pos % 4
    return jnp.where(
        k == 0, z, jnp.where(k == 1, 7 - z, jnp.where(k == 2, 8 + z, 15 - z))
    )


def _ring_to_pos(r):
    seg = r // 4
    z = jnp.where(
        seg == 0, r, jnp.where(seg == 1, 7 - r, jnp.where(seg == 2, r - 8, 15 - r))
    )
    return z * 4 + seg


def kernel(x, w_mat):
    m, k = x.shape
    k2, n = w_mat.shape
    assert k == k2 and m == N_DEV * CH

    def body(x_ref, w_ref, out_ref, acc_ref, send_sems, recv_sems):
        my_pos = lax.axis_index("i")
        my_r = _pos_to_ring(my_pos)
        right_pos = _ring_to_pos((my_r + 1) % N_DEV)
        left_pos = _ring_to_pos((my_r - 1) % N_DEV)

        barrier_sem = pltpu.get_barrier_semaphore()
        for nbr in (left_pos, right_pos):
            pl.semaphore_signal(
                barrier_sem, inc=1,
                device_id=(nbr,), device_id_type=pl.DeviceIdType.MESH,
            )
        pl.semaphore_wait(barrier_sem, 2)

        def partial_for_step(s):
            c = _ring_to_pos((my_r - s - 1) % N_DEV)
            xs = x_ref[pl.ds(c * CH, CH), :]
            return jnp.dot(xs, w_ref[...], preferred_element_type=jnp.float32)

        acc_ref[0] = partial_for_step(0)

        for s in range(N_DEV - 1):
            send_slot = s % 2
            recv_slot = (s + 1) % 2
            rdma = pltpu.make_async_remote_copy(
                src_ref=acc_ref.at[send_slot],
                dst_ref=acc_ref.at[recv_slot],
                send_sem=send_sems.at[send_slot],
                recv_sem=recv_sems.at[recv_slot],
                device_id=(right_pos,),
                device_id_type=pl.DeviceIdType.MESH,
            )
            rdma.start()
            partial = partial_for_step(s + 1)
            rdma.wait()
            if s == N_DEV - 2:
                out_ref[...] = acc_ref[recv_slot] + partial
            else:
                acc_ref[recv_slot] = acc_ref[recv_slot] + partial

    return pl.pallas_call(
        body,
        out_shape=jax.ShapeDtypeStruct((CH, n), jnp.float32),
        in_specs=[
            pl.BlockSpec(memory_space=pltpu.VMEM),
            pl.BlockSpec(memory_space=pltpu.VMEM),
        ],
        out_specs=pl.BlockSpec(memory_space=pltpu.VMEM),
        scratch_shapes=[
            pltpu.VMEM((2, CH, n), jnp.float32),
            pltpu.SemaphoreType.DMA((2,)),
            pltpu.SemaphoreType.DMA((2,)),
        ],
        compiler_params=pltpu.CompilerParams(collective_id=0),
    )(x, w_mat)


# device time: 727064 ns/iter; 1.9255x vs baseline; 1.9255x over previous
import jax
import jax.numpy as jnp
from jax import lax
from jax.experimental import pallas as pl
from jax.experimental.pallas import tpu as pltpu

N_DEV = 16
CH = 256


def _pos_to_ring(pos):
    z = pos // 4
    k = pos % 4
    return jnp.where(
        k == 0, z, jnp.where(k == 1, 7 - z, jnp.where(k == 2, 8 + z, 15 - z))
    )


def _ring_to_pos(r):
    seg = r // 4
    z = jnp.where(
        seg == 0, r, jnp.where(seg == 1, 7 - r, jnp.where(seg == 2, r - 8, 15 - r))
    )
    return z * 4 + seg


def kernel(x, w_mat):
    m, k = x.shape
    k2, n = w_mat.shape
    assert k == k2 and m == N_DEV * CH
    n2 = n // 2

    def body(x_ref, w_ref, out_ref, acc_a, acc_b,
             send_a, recv_a, send_b, recv_b):
        my_pos = lax.axis_index("i")
        my_r = _pos_to_ring(my_pos)
        right_pos = _ring_to_pos((my_r + 1) % N_DEV)
        left_pos = _ring_to_pos((my_r - 1) % N_DEV)

        barrier_sem = pltpu.get_barrier_semaphore()
        for nbr in (left_pos, right_pos):
            pl.semaphore_signal(
                barrier_sem, inc=1,
                device_id=(nbr,), device_id_type=pl.DeviceIdType.MESH,
            )
        pl.semaphore_wait(barrier_sem, 2)

        def partial_a(s):
            c = _ring_to_pos((my_r - s - 1) % N_DEV)
            xs = x_ref[pl.ds(c * CH, CH), :]
            return jnp.dot(xs, w_ref[:, :n2], preferred_element_type=jnp.float32)

        def partial_b(s):
            c = _ring_to_pos((my_r + s + 1) % N_DEV)
            xs = x_ref[pl.ds(c * CH, CH), :]
            return jnp.dot(xs, w_ref[:, n2:], preferred_element_type=jnp.float32)

        acc_a[0] = partial_a(0)
        acc_b[0] = partial_b(0)

        for s in range(N_DEV - 1):
            send_slot = s % 2
            recv_slot = (s + 1) % 2
            rdma_a = pltpu.make_async_remote_copy(
                src_ref=acc_a.at[send_slot],
                dst_ref=acc_a.at[recv_slot],
                send_sem=send_a.at[send_slot],
                recv_sem=recv_a.at[recv_slot],
                device_id=(right_pos,),
                device_id_type=pl.DeviceIdType.MESH,
            )
            rdma_b = pltpu.make_async_remote_copy(
                src_ref=acc_b.at[send_slot],
                dst_ref=acc_b.at[recv_slot],
                send_sem=send_b.at[send_slot],
                recv_sem=recv_b.at[recv_slot],
                device_id=(left_pos,),
                device_id_type=pl.DeviceIdType.MESH,
            )
            rdma_a.start()
            rdma_b.start()
            pa = partial_a(s + 1)
            pb = partial_b(s + 1)
            rdma_a.wait()
            rdma_b.wait()
            if s == N_DEV - 2:
                out_ref[:, :n2] = acc_a[recv_slot] + pa
                out_ref[:, n2:] = acc_b[recv_slot] + pb
            else:
                acc_a[recv_slot] = acc_a[recv_slot] + pa
                acc_b[recv_slot] = acc_b[recv_slot] + pb

    return pl.pallas_call(
        body,
        out_shape=jax.ShapeDtypeStruct((CH, n), jnp.float32),
        in_specs=[
            pl.BlockSpec(memory_space=pltpu.VMEM),
            pl.BlockSpec(memory_space=pltpu.VMEM),
        ],
        out_specs=pl.BlockSpec(memory_space=pltpu.VMEM),
        scratch_shapes=[
            pltpu.VMEM((2, CH, n2), jnp.float32),
            pltpu.VMEM((2, CH, n2), jnp.float32),
            pltpu.SemaphoreType.DMA((2,)),
            pltpu.SemaphoreType.DMA((2,)),
            pltpu.SemaphoreType.DMA((2,)),
            pltpu.SemaphoreType.DMA((2,)),
        ],
        compiler_params=pltpu.CompilerParams(collective_id=0),
    )(x, w_mat)


# device time: 691167 ns/iter; 2.0255x vs baseline; 1.0519x over previous
import jax
import jax.numpy as jnp
from jax import lax
from jax.experimental import pallas as pl
from jax.experimental.pallas import tpu as pltpu

N_DEV = 16
CH = 256


def _pos_to_ring(pos):
    z = pos // 4
    k = pos % 4
    return jnp.where(
        k == 0, z, jnp.where(k == 1, 7 - z, jnp.where(k == 2, 8 + z, 15 - z))
    )


def _ring_to_pos(r):
    seg = r // 4
    z = jnp.where(
        seg == 0, r, jnp.where(seg == 1, 7 - r, jnp.where(seg == 2, r - 8, 15 - r))
    )
    return z * 4 + seg


def kernel(x, w_mat):
    m, k = x.shape
    k2, n = w_mat.shape
    assert k == k2 and m == N_DEV * CH
    NF = 4
    nf = n // NF
    dirs = (+1, +1, -1, -1)

    def body(x_ref, w_ref, out_ref, *scratch):
        accs = scratch[:NF]
        send_sems = scratch[NF:2 * NF]
        recv_sems = scratch[2 * NF:3 * NF]

        my_pos = lax.axis_index("i")
        my_r = _pos_to_ring(my_pos)
        right_pos = _ring_to_pos((my_r + 1) % N_DEV)
        left_pos = _ring_to_pos((my_r - 1) % N_DEV)

        barrier_sem = pltpu.get_barrier_semaphore()
        for nbr in (left_pos, right_pos):
            pl.semaphore_signal(
                barrier_sem, inc=1,
                device_id=(nbr,), device_id_type=pl.DeviceIdType.MESH,
            )
        pl.semaphore_wait(barrier_sem, 2)

        def partial(f, s):
            c = _ring_to_pos((my_r - dirs[f] * (s + 1)) % N_DEV)
            xs = x_ref[pl.ds(c * CH, CH), :]
            return jnp.dot(
                xs, w_ref[:, f * nf:(f + 1) * nf],
                preferred_element_type=jnp.float32,
            )

        def make_rdma(f, s):
            return pltpu.make_async_remote_copy(
                src_ref=accs[f].at[s % 2],
                dst_ref=accs[f].at[(s + 1) % 2],
                send_sem=send_sems[f].at[s % 2],
                recv_sem=recv_sems[f].at[(s + 1) % 2],
                device_id=(right_pos if dirs[f] > 0 else left_pos,),
                device_id_type=pl.DeviceIdType.MESH,
            )

        for f in range(NF):
            accs[f][0] = partial(f, 0)
        for f in range(NF):
            make_rdma(f, 0).start()
        p = [partial(f, 1) for f in range(NF)]

        for s in range(1, N_DEV):
            slot = s % 2
            for f in range(NF):
                make_rdma(f, s - 1).wait()
                if s == N_DEV - 1:
                    out_ref[:, f * nf:(f + 1) * nf] = accs[f][slot] + p[f]
                else:
                    accs[f][slot] = accs[f][slot] + p[f]
                    make_rdma(f, s).start()
            if s < N_DEV - 1:
                p = [partial(f, s + 1) for f in range(NF)]

    return pl.pallas_call(
        body,
        out_shape=jax.ShapeDtypeStruct((CH, n), jnp.float32),
        in_specs=[
            pl.BlockSpec(memory_space=pltpu.VMEM),
            pl.BlockSpec(memory_space=pltpu.VMEM),
        ],
        out_specs=pl.BlockSpec(memory_space=pltpu.VMEM),
        scratch_shapes=(
            [pltpu.VMEM((2, CH, nf), jnp.float32) for _ in range(NF)]
            + [pltpu.SemaphoreType.DMA((2,)) for _ in range(NF)]
            + [pltpu.SemaphoreType.DMA((2,)) for _ in range(NF)]
        ),
        compiler_params=pltpu.CompilerParams(collective_id=0),
    )(x, w_mat)


# device time: 690404 ns/iter; 2.0277x vs baseline; 1.0011x over previous
import jax
import jax.numpy as jnp
from jax import lax
from jax.experimental import pallas as pl
from jax.experimental.pallas import tpu as pltpu

N_DEV = 16
CH = 256


def _pos_to_ring(pos):
    z = pos // 4
    k = pos % 4
    return jnp.where(
        k == 0, z, jnp.where(k == 1, 7 - z, jnp.where(k == 2, 8 + z, 15 - z))
    )


def _ring_to_pos(r):
    seg = r // 4
    z = jnp.where(
        seg == 0, r, jnp.where(seg == 1, 7 - r, jnp.where(seg == 2, r - 8, 15 - r))
    )
    return z * 4 + seg


def kernel(x, w_mat):
    m, k = x.shape
    k2, n = w_mat.shape
    assert k == k2 and m == N_DEV * CH
    NF = 4
    nf = n // NF
    dirs = (+1, +1, -1, -1)

    def body(x_ref, w_ref, out_ref, *scratch):
        accs = scratch[:NF]
        send_sems = scratch[NF:2 * NF]
        recv_sems = scratch[2 * NF:3 * NF]

        my_pos = lax.axis_index("i")
        my_r = _pos_to_ring(my_pos)
        right_pos = _ring_to_pos((my_r + 1) % N_DEV)
        left_pos = _ring_to_pos((my_r - 1) % N_DEV)

        barrier_sem = pltpu.get_barrier_semaphore()
        for nbr in (left_pos, right_pos):
            pl.semaphore_signal(
                barrier_sem, inc=1,
                device_id=(nbr,), device_id_type=pl.DeviceIdType.MESH,
            )
        pl.semaphore_wait(barrier_sem, 2)

        def partial(f, s):
            c = _ring_to_pos((my_r - dirs[f] * (s + 1)) % N_DEV)
            xs = x_ref[pl.ds(c * CH, CH), :]
            return jnp.dot(
                xs, w_ref[:, f * nf:(f + 1) * nf],
                preferred_element_type=jnp.float32,
            )

        def make_rdma(f, s):
            return pltpu.make_async_remote_copy(
                src_ref=accs[f].at[s % 2],
                dst_ref=accs[f].at[(s + 1) % 2],
                send_sem=send_sems[f].at[s % 2],
                recv_sem=recv_sems[f].at[(s + 1) % 2],
                device_id=(right_pos if dirs[f] > 0 else left_pos,),
                device_id_type=pl.DeviceIdType.MESH,
            )

        for f in range(NF):
            accs[f][0] = partial(f, 0)
            make_rdma(f, 0).start()
        p = [partial(f, 1) for f in range(NF)]

        for s in range(1, N_DEV):
            slot = s % 2
            for f in range(NF):
                make_rdma(f, s - 1).wait()
                if s == N_DEV - 1:
                    out_ref[:, f * nf:(f + 1) * nf] = accs[f][slot] + p[f]
                else:
                    accs[f][slot] = accs[f][slot] + p[f]
                    make_rdma(f, s).start()
            if s < N_DEV - 1:
                p = [partial(f, s + 1) for f in range(NF)]

    return pl.pallas_call(
        body,
        out_shape=jax.ShapeDtypeStruct((CH, n), jnp.float32),
        in_specs=[
            pl.BlockSpec(memory_space=pltpu.VMEM),
            pl.BlockSpec(memory_space=pltpu.VMEM),
        ],
        out_specs=pl.BlockSpec(memory_space=pltpu.VMEM),
        scratch_shapes=(
            [pltpu.VMEM((2, CH, nf), jnp.float32) for _ in range(NF)]
            + [pltpu.SemaphoreType.DMA((2,)) for _ in range(NF)]
            + [pltpu.SemaphoreType.DMA((2,)) for _ in range(NF)]
        ),
        compiler_params=pltpu.CompilerParams(collective_id=0),
    )(x, w_mat)
